# Initial kernel scaffold; baseline (speedup 1.0000x reference)
import jax
import jax.numpy as jnp
from jax.experimental import pallas as pl


def kernel(x, expert_W, expert_b, router_W, router_b):
    raise NotImplementedError("write your pallas kernel here")


# speedup vs baseline: 1.2984x; 1.2984x over previous
import jax
import jax.numpy as jnp
from jax.experimental import pallas as pl
from jax.experimental.pallas import tpu as pltpu

NUM_EXPERTS = 8
ROW_BLOCK = 1024


def _moe_dense_body(x_ref, rw_ref, rb_ref, w_ref, b_ref, out_ref,
                    scale_ref, expertf_ref):
    e = pl.program_id(1)

    @pl.when(e == 0)
    def _():
        xb = x_ref[...]
        logits = jax.lax.dot_general(
            xb, rw_ref[...], (((1,), (1,)), ((), ())),
            preferred_element_type=jnp.float32) + rb_ref[...]
        m = jnp.max(logits, axis=1, keepdims=True)
        sumexp = jnp.sum(jnp.exp(logits - m), axis=1, keepdims=True)
        scale_ref[...] = 1.0 / sumexp
        iota = jax.lax.broadcasted_iota(jnp.int32, logits.shape, 1
                                        ).astype(jnp.float32)
        expertf_ref[...] = jnp.min(
            jnp.where(logits >= m, iota, jnp.float32(NUM_EXPERTS)),
            axis=1, keepdims=True)

    xb = x_ref[...]
    y = jax.lax.dot_general(
        xb, w_ref[0], (((1,), (1,)), ((), ())),
        preferred_element_type=jnp.float32) + b_ref[0]
    w = jnp.where(expertf_ref[...] == jnp.float32(e), scale_ref[...], 0.0)
    contrib = y * w

    @pl.when(e == 0)
    def _():
        out_ref[...] = contrib

    @pl.when(e != 0)
    def _():
        out_ref[...] += contrib


def kernel(x, expert_W, expert_b, router_W, router_b):
    B, C, D = x.shape
    N = B * C
    E = expert_W.shape[0]
    xf = x.reshape(N, D)

    grid = (N // ROW_BLOCK, E)
    out = pl.pallas_call(
        _moe_dense_body,
        grid=grid,
        in_specs=[
            pl.BlockSpec((ROW_BLOCK, D), lambda i, e: (i, 0)),
            pl.BlockSpec((E, D), lambda i, e: (0, 0)),
            pl.BlockSpec((1, E), lambda i, e: (0, 0)),
            pl.BlockSpec((1, D, D), lambda i, e: (e, 0, 0)),
            pl.BlockSpec((1, 1, D), lambda i, e: (e, 0, 0)),
        ],
        out_specs=pl.BlockSpec((ROW_BLOCK, D), lambda i, e: (i, 0)),
        out_shape=jax.ShapeDtypeStruct((N, D), jnp.float32),
        scratch_shapes=[
            pltpu.VMEM((ROW_BLOCK, 1), jnp.float32),
            pltpu.VMEM((ROW_BLOCK, 1), jnp.float32),
        ],
    )(xf, router_W, router_b.reshape(1, E), expert_W,
      expert_b.reshape(E, 1, D))

    return out.reshape(B, C, D), 0
